# baseline (speedup 1.0000x reference)
import functools

import jax
import jax.numpy as jnp
from jax import lax
from jax.experimental import pallas as pl
from jax.experimental.pallas import tpu as pltpu
from jax.experimental.pallas import tpu_sc as plsc

VOCAB = 1000
D_MODEL = 128
B = 1024
L = 200
N_TOK = B * L

_NC = 2
_NS = 16
_NW = _NC * _NS
_PER_W = N_TOK // _NW
_CHUNK = 64
_NCHUNK = _PER_W // _CHUNK


def _table_body(emb_ref, w1_ref, b1_ref, w2_ref, b2_ref, out_ref):
    h = jnp.dot(emb_ref[:], w1_ref[:], preferred_element_type=jnp.float32)
    h = jnp.maximum(h + b1_ref[:], 0.0)
    out_ref[:] = (
        jnp.dot(h, w2_ref[:], preferred_element_type=jnp.float32) + b2_ref[:]
    )


def _make_table(emb, W1, b1, W2, b2):
    return pl.pallas_call(
        _table_body,
        out_shape=jax.ShapeDtypeStruct((VOCAB, VOCAB), jnp.float32),
    )(emb, W1, b1.reshape(1, D_MODEL), W2, b2.reshape(1, VOCAB))


def _gather_body(table_hbm, idx_hbm, out_hbm, idx_v, rows_v, sem):
    wid = lax.axis_index("s") * _NC + lax.axis_index("c")
    base = wid * _PER_W

    def step(i, carry):
        off = base + i * _CHUNK
        pltpu.sync_copy(idx_hbm.at[pl.ds(off, _CHUNK)], idx_v)
        pltpu.async_copy(table_hbm.at[idx_v], rows_v, sem).wait()
        pltpu.sync_copy(rows_v, out_hbm.at[pl.ds(off, _CHUNK)])
        return carry

    lax.fori_loop(0, _NCHUNK, step, 0)


_gather = functools.partial(
    pl.kernel,
    out_type=jax.ShapeDtypeStruct((N_TOK, VOCAB), jnp.float32),
    mesh=plsc.VectorSubcoreMesh(core_axis_name="c", subcore_axis_name="s"),
    scratch_types=[
        pltpu.VMEM((_CHUNK,), jnp.int32),
        pltpu.VMEM((_CHUNK, VOCAB), jnp.float32),
        pltpu.SemaphoreType.DMA,
    ],
    compiler_params=pltpu.CompilerParams(use_tc_tiling_on_sc=False),
)(_gather_body)


def kernel(idx, emb, W1, b1, W2, b2):
    table = _make_table(emb, W1, b1, W2, b2)
    flat_idx = idx.reshape(N_TOK).astype(jnp.int32)
    out = _gather(table, flat_idx)
    return out.reshape(B, L, VOCAB)


# speedup vs baseline: 1.0371x; 1.0371x over previous
import functools

import jax
import jax.numpy as jnp
from jax import lax
from jax.experimental import pallas as pl
from jax.experimental.pallas import tpu as pltpu
from jax.experimental.pallas import tpu_sc as plsc

VOCAB = 1000
D_MODEL = 128
B = 1024
L = 200
N_TOK = B * L

_NC = 2
_NS = 16
_NW = _NC * _NS
_PER_W = N_TOK // _NW
_CHUNK = 40
_NCHUNK = _PER_W // _CHUNK


def _table_body(emb_ref, w1_ref, b1_ref, w2_ref, b2_ref, out_ref):
    h = jnp.dot(emb_ref[:], w1_ref[:], preferred_element_type=jnp.float32)
    h = jnp.maximum(h + b1_ref[:], 0.0)
    out_ref[:] = (
        jnp.dot(h, w2_ref[:], preferred_element_type=jnp.float32) + b2_ref[:]
    )


def _make_table(emb, W1, b1, W2, b2):
    return pl.pallas_call(
        _table_body,
        out_shape=jax.ShapeDtypeStruct((VOCAB, VOCAB), jnp.float32),
    )(emb, W1, b1.reshape(1, D_MODEL), W2, b2.reshape(1, VOCAB))


def _gather_body(table_hbm, idx_hbm, out_hbm, idx_v, rows_v, g0, g1, s0, s1):
    wid = lax.axis_index("s") * _NC + lax.axis_index("c")
    base = wid * _PER_W
    gsem = (g0, g1)
    ssem = (s0, s1)

    def idx_at(i):
        return idx_v.at[pl.ds(i * _CHUNK, _CHUNK)]

    def gather_start(b, i):
        pltpu.async_copy(table_hbm.at[idx_at(i)], rows_v.at[b], gsem[b])

    def gather_wait(b):
        pltpu.make_async_copy(
            table_hbm.at[pl.ds(0, _CHUNK)], rows_v.at[b], gsem[b]
        ).wait()

    def scatter_start(b, i):
        pltpu.async_copy(
            rows_v.at[b], out_hbm.at[pl.ds(base + i * _CHUNK, _CHUNK)], ssem[b]
        )

    def scatter_wait(b):
        pltpu.make_async_copy(
            rows_v.at[b], out_hbm.at[pl.ds(base, _CHUNK)], ssem[b]
        ).wait()

    pltpu.sync_copy(idx_hbm.at[pl.ds(base, _PER_W)], idx_v)

    gather_start(0, 0)
    gather_start(1, 1)
    gather_wait(0)
    scatter_start(0, 0)

    def step(g, carry):
        scatter_wait(0)
        gather_start(0, 2 * g)
        gather_wait(1)
        scatter_start(1, 2 * g - 1)
        scatter_wait(1)
        gather_start(1, 2 * g + 1)
        gather_wait(0)
        scatter_start(0, 2 * g)
        return carry

    lax.fori_loop(1, _NCHUNK // 2, step, 0)

    gather_wait(1)
    scatter_start(1, _NCHUNK - 1)
    scatter_wait(0)
    scatter_wait(1)


_gather = functools.partial(
    pl.kernel,
    out_type=jax.ShapeDtypeStruct((N_TOK, VOCAB), jnp.float32),
    mesh=plsc.VectorSubcoreMesh(core_axis_name="c", subcore_axis_name="s"),
    scratch_types=[
        pltpu.VMEM((_PER_W,), jnp.int32),
        pltpu.VMEM((2, _CHUNK, VOCAB), jnp.float32),
        pltpu.SemaphoreType.DMA,
        pltpu.SemaphoreType.DMA,
        pltpu.SemaphoreType.DMA,
        pltpu.SemaphoreType.DMA,
    ],
    compiler_params=pltpu.CompilerParams(use_tc_tiling_on_sc=False),
)(_gather_body)


def kernel(idx, emb, W1, b1, W2, b2):
    table = _make_table(emb, W1, b1, W2, b2)
    flat_idx = idx.reshape(N_TOK).astype(jnp.int32)
    out = _gather(table, flat_idx)
    return out.reshape(B, L, VOCAB)


# speedup vs baseline: 1.1411x; 1.1003x over previous
import functools

import jax
import jax.numpy as jnp
from jax import lax
from jax.experimental import pallas as pl
from jax.experimental.pallas import tpu as pltpu
from jax.experimental.pallas import tpu_sc as plsc

VOCAB = 1000
D_MODEL = 128
B = 1024
L = 200
N_TOK = B * L

_VPAD = 1024
_NC = 2
_NS = 16
_NW = _NC * _NS
_PER_W = N_TOK // _NW
_CHUNK = 40
_NCHUNK = _PER_W // _CHUNK


def _table_body(emb_ref, w1_ref, b1_ref, w2_ref, b2_ref, out_ref):
    h = jnp.dot(emb_ref[:], w1_ref[:], preferred_element_type=jnp.float32)
    h = jnp.maximum(h + b1_ref[:], 0.0)
    t = jnp.dot(h, w2_ref[:], preferred_element_type=jnp.float32) + b2_ref[:]
    out_ref[:] = jnp.pad(t, ((0, 0), (0, _VPAD - VOCAB)))


def _make_table(emb, W1, b1, W2, b2):
    return pl.pallas_call(
        _table_body,
        out_shape=jax.ShapeDtypeStruct((VOCAB, _VPAD), jnp.float32),
    )(emb, W1, b1.reshape(1, D_MODEL), W2, b2.reshape(1, VOCAB))


def _gather_body(table_hbm, idx_hbm, out_hbm, idx_v, raw_v, rows_v, gsem, s0, s1):
    wid = lax.axis_index("s") * _NC + lax.axis_index("c")
    base = wid * _PER_W
    ssem = (s0, s1)

    def gather_start(i):
        pltpu.async_copy(
            table_hbm.at[idx_v.at[pl.ds(i * _CHUNK, _CHUNK)]], raw_v, gsem
        )

    def gather_wait():
        pltpu.make_async_copy(
            table_hbm.at[pl.ds(0, _CHUNK)], raw_v, gsem
        ).wait()

    def merge(b):
        def row(j, carry):
            for k in range(62):
                rows_v[b, j, pl.ds(16 * k, 16)] = raw_v[j, pl.ds(16 * k, 16)]
            rows_v[b, j, pl.ds(VOCAB - 16, 16)] = raw_v[j, pl.ds(VOCAB - 16, 16)]
            return carry

        lax.fori_loop(0, _CHUNK, row, 0)

    def scatter_start(b, i):
        pltpu.async_copy(
            rows_v.at[b], out_hbm.at[pl.ds(base + i * _CHUNK, _CHUNK)], ssem[b]
        )

    def scatter_wait(b):
        pltpu.make_async_copy(
            rows_v.at[b], out_hbm.at[pl.ds(base, _CHUNK)], ssem[b]
        ).wait()

    pltpu.sync_copy(idx_hbm.at[pl.ds(base, _PER_W)], idx_v)

    gather_start(0)
    gather_wait()
    merge(0)
    gather_start(1)
    scatter_start(0, 0)
    gather_wait()
    merge(1)
    gather_start(2)
    scatter_start(1, 1)

    def step(g, carry):
        gather_wait()
        merge(0)
        gather_start(2 * g + 1)
        scatter_wait(0)
        scatter_start(0, 2 * g)
        gather_wait()
        merge(1)
        gather_start(2 * g + 2)
        scatter_wait(1)
        scatter_start(1, 2 * g + 1)
        return carry

    lax.fori_loop(1, _NCHUNK // 2 - 1, step, 0)

    gather_wait()
    merge(0)
    gather_start(_NCHUNK - 1)
    scatter_wait(0)
    scatter_start(0, _NCHUNK - 2)
    gather_wait()
    merge(1)
    scatter_wait(1)
    scatter_start(1, _NCHUNK - 1)
    scatter_wait(0)
    scatter_wait(1)


_gather = functools.partial(
    pl.kernel,
    out_type=jax.ShapeDtypeStruct((N_TOK, VOCAB), jnp.float32),
    mesh=plsc.VectorSubcoreMesh(core_axis_name="c", subcore_axis_name="s"),
    scratch_types=[
        pltpu.VMEM((_PER_W,), jnp.int32),
        pltpu.VMEM((_CHUNK, _VPAD), jnp.float32),
        pltpu.VMEM((2, _CHUNK, VOCAB), jnp.float32),
        pltpu.SemaphoreType.DMA,
        pltpu.SemaphoreType.DMA,
        pltpu.SemaphoreType.DMA,
    ],
)(_gather_body)


def kernel(idx, emb, W1, b1, W2, b2):
    table = _make_table(emb, W1, b1, W2, b2)
    flat_idx = idx.reshape(N_TOK).astype(jnp.int32)
    out = _gather(table, flat_idx)
    return out.reshape(B, L, VOCAB)


# speedup vs baseline: 1.2103x; 1.0607x over previous
import functools

import jax
import jax.numpy as jnp
from jax import lax
from jax.experimental import pallas as pl
from jax.experimental.pallas import tpu as pltpu
from jax.experimental.pallas import tpu_sc as plsc

VOCAB = 1000
D_MODEL = 128
B = 1024
L = 200
N_TOK = B * L

_VPAD = 1024
_NC = 2
_NS = 16
_NW = _NC * _NS
_VSTRIP = _VPAD // _NW
_LANES = 16


def _table_body(emb_ref, w1_ref, b1_ref, w2_ref, b2_ref, out_ref):
    h = jnp.dot(emb_ref[:], w1_ref[:], preferred_element_type=jnp.float32)
    h = jnp.maximum(h + b1_ref[:], 0.0)
    w2p = jnp.pad(w2_ref[:], ((0, 0), (0, _VPAD - VOCAB)))
    b2p = jnp.pad(b2_ref[:], ((0, 0), (0, _VPAD - VOCAB)))
    t_t = (
        jax.lax.dot_general(
            w2p, h, (((0,), (1,)), ((), ())),
            preferred_element_type=jnp.float32,
        )
        + b2p.reshape(_VPAD, 1)
    )
    out_ref[:] = t_t


def _make_table_t(emb, W1, b1, W2, b2):
    return pl.pallas_call(
        _table_body,
        out_shape=jax.ShapeDtypeStruct((_VPAD, VOCAB), jnp.float32),
    )(emb, W1, b1.reshape(1, D_MODEL), W2, b2.reshape(1, VOCAB))


def _expand_body(tt_hbm, idx_hbm, out_hbm, ttab, idxb, stage, i0, i1, s0, s1):
    wid = lax.axis_index("s") * _NC + lax.axis_index("c")
    v0 = wid * _VSTRIP
    isem = (i0, i1)
    ssem = (s0, s1)
    last = _NW - 1
    nvalid = VOCAB - _VSTRIP * last

    pltpu.sync_copy(tt_hbm.at[pl.ds(v0, _VSTRIP)], ttab)

    def idx_start(bi, l):
        pltpu.async_copy(
            idx_hbm.at[pl.ds(l, 1)], idxb.at[pl.ds(bi, 1)], isem[bi]
        )

    def idx_wait(bi):
        pltpu.make_async_copy(
            idx_hbm.at[pl.ds(0, 1)], idxb.at[pl.ds(bi, 1)], isem[bi]
        ).wait()

    def compute(bi, sb):
        def bblk(t, carry):
            base = t * (4 * _LANES)
            ivs = [
                idxb[bi, pl.ds(base + _LANES * k, _LANES)] for k in range(4)
            ]
            for vl in range(_VSTRIP):
                vsplat = jnp.full((_LANES,), vl, jnp.int32)
                for k in range(4):
                    stage[sb, vl, pl.ds(base + _LANES * k, _LANES)] = (
                        plsc.load_gather(ttab, [vsplat, ivs[k]])
                    )
            return carry

        lax.fori_loop(0, B // (4 * _LANES), bblk, 0)

    def scatter_start(sb, l):
        @pl.when(wid < last)
        def _():
            pltpu.async_copy(
                stage.at[pl.ds(sb, 1)],
                out_hbm.at[pl.ds(l, 1), pl.ds(v0, _VSTRIP)],
                ssem[sb],
            )

        @pl.when(wid == last)
        def _():
            pltpu.async_copy(
                stage.at[pl.ds(sb, 1), pl.ds(0, nvalid)],
                out_hbm.at[pl.ds(l, 1), pl.ds(v0, nvalid)],
                ssem[sb],
            )

    def scatter_wait(sb):
        @pl.when(wid < last)
        def _():
            pltpu.make_async_copy(
                stage.at[pl.ds(sb, 1)],
                out_hbm.at[pl.ds(0, 1), pl.ds(v0, _VSTRIP)],
                ssem[sb],
            ).wait()

        @pl.when(wid == last)
        def _():
            pltpu.make_async_copy(
                stage.at[pl.ds(sb, 1), pl.ds(0, nvalid)],
                out_hbm.at[pl.ds(0, 1), pl.ds(v0, nvalid)],
                ssem[sb],
            ).wait()

    idx_start(0, 0)
    idx_start(1, 1)
    idx_wait(0)
    compute(0, 0)
    scatter_start(0, 0)
    idx_start(0, 2)
    idx_wait(1)
    compute(1, 1)
    scatter_start(1, 1)
    idx_start(1, 3)

    def step(lp, carry):
        l0 = 2 * lp
        idx_wait(0)
        scatter_wait(0)
        compute(0, 0)
        scatter_start(0, l0)
        idx_start(0, l0 + 2)
        idx_wait(1)
        scatter_wait(1)
        compute(1, 1)
        scatter_start(1, l0 + 1)
        idx_start(1, l0 + 3)
        return carry

    lax.fori_loop(1, L // 2 - 1, step, 0)

    idx_wait(0)
    scatter_wait(0)
    compute(0, 0)
    scatter_start(0, L - 2)
    idx_wait(1)
    scatter_wait(1)
    compute(1, 1)
    scatter_start(1, L - 1)
    scatter_wait(0)
    scatter_wait(1)


_expand = functools.partial(
    pl.kernel,
    out_type=jax.ShapeDtypeStruct((L, VOCAB, B), jnp.float32),
    mesh=plsc.VectorSubcoreMesh(core_axis_name="c", subcore_axis_name="s"),
    scratch_types=[
        pltpu.VMEM((_VSTRIP, VOCAB), jnp.float32),
        pltpu.VMEM((2, B), jnp.int32),
        pltpu.VMEM((2, _VSTRIP, B), jnp.float32),
        pltpu.SemaphoreType.DMA,
        pltpu.SemaphoreType.DMA,
        pltpu.SemaphoreType.DMA,
        pltpu.SemaphoreType.DMA,
    ],
    compiler_params=pltpu.CompilerParams(needs_layout_passes=False),
)(_expand_body)


def kernel(idx, emb, W1, b1, W2, b2):
    table_t = _make_table_t(emb, W1, b1, W2, b2)
    idx_t = jnp.transpose(idx.astype(jnp.int32))
    out_phys = _expand(table_t, idx_t)
    return jnp.transpose(out_phys, (2, 0, 1))


# speedup vs baseline: 2.7316x; 2.2569x over previous
import functools

import jax
import jax.numpy as jnp
from jax import lax
from jax.experimental import pallas as pl
from jax.experimental.pallas import tpu as pltpu
from jax.experimental.pallas import tpu_sc as plsc

VOCAB = 1000
D_MODEL = 128
B = 1024
L = 200
N_TOK = B * L

_VPAD = 1024
_NC = 2
_NS = 16
_NW = _NC * _NS
_VSTRIP = _VPAD // _NW
_LANES = 16


def _table_body(emb_ref, w1_ref, b1_ref, w2_ref, b2_ref, out_ref):
    h = jnp.dot(emb_ref[:], w1_ref[:], preferred_element_type=jnp.float32)
    h = jnp.maximum(h + b1_ref[:], 0.0)
    w2p = jnp.pad(w2_ref[:], ((0, 0), (0, _VPAD - VOCAB)))
    b2p = jnp.pad(b2_ref[:], ((0, 0), (0, _VPAD - VOCAB)))
    t_t = (
        jax.lax.dot_general(
            w2p, h, (((0,), (1,)), ((), ())),
            preferred_element_type=jnp.float32,
        )
        + b2p.reshape(_VPAD, 1)
    )
    out_ref[:] = jnp.pad(t_t, ((0, 0), (0, _VPAD - VOCAB)))


def _make_table_t(emb, W1, b1, W2, b2):
    return pl.pallas_call(
        _table_body,
        out_shape=jax.ShapeDtypeStruct((_VPAD, _VPAD), jnp.float32),
    )(emb, W1, b1.reshape(1, D_MODEL), W2, b2.reshape(1, VOCAB))


def _expand_body(tt_hbm, idx_hbm, out_hbm, ttab, idxb, stage, i0, i1, s0, s1):
    wid = lax.axis_index("s") * _NC + lax.axis_index("c")
    v0 = wid * _VSTRIP
    isem = (i0, i1)
    ssem = (s0, s1)
    last = _NW - 1
    nvalid = VOCAB - _VSTRIP * last

    pltpu.sync_copy(tt_hbm.at[pl.ds(v0 * _VPAD, _VSTRIP * _VPAD)], ttab)

    def idx_start(bi, l):
        pltpu.async_copy(
            idx_hbm.at[pl.ds(l, 1)], idxb.at[pl.ds(bi, 1)], isem[bi]
        )

    def idx_wait(bi):
        pltpu.make_async_copy(
            idx_hbm.at[pl.ds(0, 1)], idxb.at[pl.ds(bi, 1)], isem[bi]
        ).wait()

    def compute(bi, sb):
        def bblk(t, carry):
            base = t * (4 * _LANES)
            ivs = [
                idxb[bi, pl.ds(base + _LANES * k, _LANES)] for k in range(4)
            ]
            for vl in range(_VSTRIP):
                row = ttab.at[pl.ds(vl * _VPAD, _VPAD)]
                gs = [plsc.load_gather(row, [ivs[k]]) for k in range(4)]
                for k in range(4):
                    stage[sb, vl, pl.ds(base + _LANES * k, _LANES)] = gs[k]
            return carry

        lax.fori_loop(0, B // (4 * _LANES), bblk, 0)

    def scatter_start(sb, l):
        @pl.when(wid < last)
        def _():
            pltpu.async_copy(
                stage.at[pl.ds(sb, 1)],
                out_hbm.at[pl.ds(l, 1), pl.ds(v0, _VSTRIP)],
                ssem[sb],
            )

        @pl.when(wid == last)
        def _():
            pltpu.async_copy(
                stage.at[pl.ds(sb, 1), pl.ds(0, nvalid)],
                out_hbm.at[pl.ds(l, 1), pl.ds(v0, nvalid)],
                ssem[sb],
            )

    def scatter_wait(sb):
        @pl.when(wid < last)
        def _():
            pltpu.make_async_copy(
                stage.at[pl.ds(sb, 1)],
                out_hbm.at[pl.ds(0, 1), pl.ds(v0, _VSTRIP)],
                ssem[sb],
            ).wait()

        @pl.when(wid == last)
        def _():
            pltpu.make_async_copy(
                stage.at[pl.ds(sb, 1), pl.ds(0, nvalid)],
                out_hbm.at[pl.ds(0, 1), pl.ds(v0, nvalid)],
                ssem[sb],
            ).wait()

    idx_start(0, 0)
    idx_start(1, 1)
    idx_wait(0)
    compute(0, 0)
    scatter_start(0, 0)
    idx_start(0, 2)
    idx_wait(1)
    compute(1, 1)
    scatter_start(1, 1)
    idx_start(1, 3)

    def step(lp, carry):
        l0 = 2 * lp
        idx_wait(0)
        scatter_wait(0)
        compute(0, 0)
        scatter_start(0, l0)
        idx_start(0, l0 + 2)
        idx_wait(1)
        scatter_wait(1)
        compute(1, 1)
        scatter_start(1, l0 + 1)
        idx_start(1, l0 + 3)
        return carry

    lax.fori_loop(1, L // 2 - 1, step, 0)

    idx_wait(0)
    scatter_wait(0)
    compute(0, 0)
    scatter_start(0, L - 2)
    idx_wait(1)
    scatter_wait(1)
    compute(1, 1)
    scatter_start(1, L - 1)
    scatter_wait(0)
    scatter_wait(1)


_expand = functools.partial(
    pl.kernel,
    out_type=jax.ShapeDtypeStruct((L, VOCAB, B), jnp.float32),
    mesh=plsc.VectorSubcoreMesh(core_axis_name="c", subcore_axis_name="s"),
    scratch_types=[
        pltpu.VMEM((_VSTRIP * _VPAD,), jnp.float32),
        pltpu.VMEM((2, B), jnp.int32),
        pltpu.VMEM((2, _VSTRIP, B), jnp.float32),
        pltpu.SemaphoreType.DMA,
        pltpu.SemaphoreType.DMA,
        pltpu.SemaphoreType.DMA,
        pltpu.SemaphoreType.DMA,
    ],
    compiler_params=pltpu.CompilerParams(needs_layout_passes=False),
)(_expand_body)


def kernel(idx, emb, W1, b1, W2, b2):
    table_t = _make_table_t(emb, W1, b1, W2, b2).reshape(_VPAD * _VPAD)
    idx_t = jnp.transpose(idx.astype(jnp.int32))
    out_phys = _expand(table_t, idx_t)
    return jnp.transpose(out_phys, (2, 0, 1))


# speedup vs baseline: 3.6676x; 1.3427x over previous
import functools

import jax
import jax.numpy as jnp
from jax import lax
from jax.experimental import pallas as pl
from jax.experimental.pallas import tpu as pltpu
from jax.experimental.pallas import tpu_sc as plsc

VOCAB = 1000
D_MODEL = 128
B = 1024
L = 200
N_TOK = B * L

_VPAD = 1024
_NC = 2
_NS = 16
_NW = _NC * _NS
_VSTRIP = _VPAD // _NW
_LANES = 16


def _table_body(emb_ref, w1_ref, b1_ref, w2_ref, b2_ref, out_ref):
    h = jnp.dot(emb_ref[:], w1_ref[:], preferred_element_type=jnp.float32)
    h = jnp.maximum(h + b1_ref[:], 0.0)
    w2p = jnp.pad(w2_ref[:], ((0, 0), (0, _VPAD - VOCAB)))
    b2p = jnp.pad(b2_ref[:], ((0, 0), (0, _VPAD - VOCAB)))
    t_t = (
        jax.lax.dot_general(
            w2p, h, (((0,), (1,)), ((), ())),
            preferred_element_type=jnp.float32,
        )
        + b2p.reshape(_VPAD, 1)
    )
    out_ref[:] = jnp.pad(t_t, ((0, 0), (0, _VPAD - VOCAB)))


def _make_table_t(emb, W1, b1, W2, b2):
    return pl.pallas_call(
        _table_body,
        out_shape=jax.ShapeDtypeStruct((_VPAD, _VPAD), jnp.float32),
    )(emb, W1, b1.reshape(1, D_MODEL), W2, b2.reshape(1, VOCAB))


def _expand_body(tt_hbm, idx_hbm, out_hbm, ttab, idxb, stage, i0, i1, s0, s1):
    wid = lax.axis_index("s") * _NC + lax.axis_index("c")
    v0 = wid * _VSTRIP
    isem = (i0, i1)
    ssem = (s0, s1)
    last = _NW - 1
    nvalid = VOCAB - _VSTRIP * last

    pltpu.sync_copy(tt_hbm.at[pl.ds(v0 * _VPAD, _VSTRIP * _VPAD)], ttab)

    def idx_start(bi, l):
        pltpu.async_copy(
            idx_hbm.at[pl.ds(l, 1)], idxb.at[pl.ds(bi, 1)], isem[bi]
        )

    def idx_wait(bi):
        pltpu.make_async_copy(
            idx_hbm.at[pl.ds(0, 1)], idxb.at[pl.ds(bi, 1)], isem[bi]
        ).wait()

    def compute(bi, sb):
        def bblk(t, carry):
            base = t * (4 * _LANES)
            ivs = [
                idxb[bi, pl.ds(base + _LANES * k, _LANES)] for k in range(4)
            ]
            for vl in range(0, _VSTRIP, 2):
                row_a = ttab.at[pl.ds(vl * _VPAD, _VPAD)]
                row_b = ttab.at[pl.ds((vl + 1) * _VPAD, _VPAD)]
                ga = [plsc.load_gather(row_a, [ivs[k]]) for k in range(4)]
                gb = [plsc.load_gather(row_b, [ivs[k]]) for k in range(4)]
                for k in range(4):
                    stage[sb, vl, pl.ds(base + _LANES * k, _LANES)] = ga[k]
                for k in range(4):
                    stage[sb, vl + 1, pl.ds(base + _LANES * k, _LANES)] = gb[k]
            return carry

        lax.fori_loop(0, B // (4 * _LANES), bblk, 0)

    def scatter_start(sb, l):
        @pl.when(wid < last)
        def _():
            pltpu.async_copy(
                stage.at[pl.ds(sb, 1)],
                out_hbm.at[pl.ds(l, 1), pl.ds(v0, _VSTRIP)],
                ssem[sb],
            )

        @pl.when(wid == last)
        def _():
            pltpu.async_copy(
                stage.at[pl.ds(sb, 1), pl.ds(0, nvalid)],
                out_hbm.at[pl.ds(l, 1), pl.ds(v0, nvalid)],
                ssem[sb],
            )

    def scatter_wait(sb):
        @pl.when(wid < last)
        def _():
            pltpu.make_async_copy(
                stage.at[pl.ds(sb, 1)],
                out_hbm.at[pl.ds(0, 1), pl.ds(v0, _VSTRIP)],
                ssem[sb],
            ).wait()

        @pl.when(wid == last)
        def _():
            pltpu.make_async_copy(
                stage.at[pl.ds(sb, 1), pl.ds(0, nvalid)],
                out_hbm.at[pl.ds(0, 1), pl.ds(v0, nvalid)],
                ssem[sb],
            ).wait()

    idx_start(0, 0)
    idx_start(1, 1)
    idx_wait(0)
    compute(0, 0)
    scatter_start(0, 0)
    idx_start(0, 2)
    idx_wait(1)
    compute(1, 1)
    scatter_start(1, 1)
    idx_start(1, 3)

    def step(lp, carry):
        l0 = 2 * lp
        idx_wait(0)
        scatter_wait(0)
        compute(0, 0)
        scatter_start(0, l0)
        idx_start(0, l0 + 2)
        idx_wait(1)
        scatter_wait(1)
        compute(1, 1)
        scatter_start(1, l0 + 1)
        idx_start(1, l0 + 3)
        return carry

    lax.fori_loop(1, L // 2 - 1, step, 0)

    idx_wait(0)
    scatter_wait(0)
    compute(0, 0)
    scatter_start(0, L - 2)
    idx_wait(1)
    scatter_wait(1)
    compute(1, 1)
    scatter_start(1, L - 1)
    scatter_wait(0)
    scatter_wait(1)


_expand = functools.partial(
    pl.kernel,
    out_type=jax.ShapeDtypeStruct((L, VOCAB, B), jnp.float32),
    mesh=plsc.VectorSubcoreMesh(core_axis_name="c", subcore_axis_name="s"),
    scratch_types=[
        pltpu.VMEM((_VSTRIP * _VPAD,), jnp.float32),
        pltpu.VMEM((2, B), jnp.int32),
        pltpu.VMEM((2, _VSTRIP, B), jnp.float32),
        pltpu.SemaphoreType.DMA,
        pltpu.SemaphoreType.DMA,
        pltpu.SemaphoreType.DMA,
        pltpu.SemaphoreType.DMA,
    ],
    compiler_params=pltpu.CompilerParams(needs_layout_passes=False),
)(_expand_body)


def kernel(idx, emb, W1, b1, W2, b2):
    table_t = _make_table_t(emb, W1, b1, W2, b2).reshape(_VPAD * _VPAD)
    idx_t = jnp.transpose(idx.astype(jnp.int32))
    out_phys = _expand(table_t, idx_t)
    return jnp.transpose(out_phys, (2, 0, 1))
